# baseline (speedup 1.0000x reference)
import functools
import jax
import jax.numpy as jnp
from jax import lax
from jax.experimental import pallas as pl
from jax.experimental.pallas import tpu as pltpu
from jax.experimental.pallas import tpu_sc as plsc

_B, _N, _M, _K = 2, 2048, 1024, 5
_TILE = 1024
_T = _N // _TILE
_INF = 3.0e38
_SELF = 1.0e38

_NSUB = 32
_QPW = (_B * _N) // _NSUB
_GRP = _QPW // 16


def _sc_align_body(xh_hbm, x_hbm, out_hbm, qx_v, qy_v, qz_v,
                   cx_v, cy_v, cz_v, cn_v, acc_v, sem):
    c = lax.axis_index("c")
    s = lax.axis_index("s")
    wid = s * 2 + c
    b = wid // (_NSUB // _B)
    chunk = wid % (_NSUB // _B)
    q0 = pl.multiple_of(chunk * _QPW, 8)

    cps = [
        pltpu.async_copy(xh_hbm.at[b * 3 + 0, 0, pl.ds(q0, _QPW)], qx_v, sem),
        pltpu.async_copy(xh_hbm.at[b * 3 + 1, 0, pl.ds(q0, _QPW)], qy_v, sem),
        pltpu.async_copy(xh_hbm.at[b * 3 + 2, 0, pl.ds(q0, _QPW)], qz_v, sem),
        pltpu.async_copy(x_hbm.at[b * 3 + 0, 0], cx_v, sem),
        pltpu.async_copy(x_hbm.at[b * 3 + 1, 0], cy_v, sem),
        pltpu.async_copy(x_hbm.at[b * 3 + 2, 0], cz_v, sem),
    ]
    for cp in cps:
        cp.wait()

    def prep(k, carry):
        sl = pl.ds(k * 16, 16)
        cx = cx_v[sl]
        cy = cy_v[sl]
        cz = cz_v[sl]
        cn_v[sl] = (cx * cx + cy * cy) + cz * cz
        cx_v[sl] = -2.0 * cx
        cy_v[sl] = -2.0 * cy
        cz_v[sl] = -2.0 * cz
        return carry

    lax.fori_loop(0, _M // 16, prep, 0)

    for g in range(_GRP):
        qx = qx_v[pl.ds(g * 16, 16)]
        qy = qy_v[pl.ds(g * 16, 16)]
        qz = qz_v[pl.ds(g * 16, 16)]
        qn = (qx * qx + qy * qy) + qz * qz

        init = tuple(jnp.full((16,), _INF, jnp.float32) for _ in range(_K))

        def body(jj, ts, qx=qx, qy=qy, qz=qz):
            for u in range(4):
                j = jj * 4 + u
                idx = jnp.full((16,), j, jnp.int32)
                gx = plsc.load_gather(cx_v, [idx])
                gy = plsc.load_gather(cy_v, [idx])
                gz = plsc.load_gather(cz_v, [idx])
                gn = plsc.load_gather(cn_v, [idx])
                d = (gn + qx * gx) + (qy * gy + qz * gz)
                new = [jnp.minimum(ts[0], d)]
                for i in range(1, _K):
                    new.append(jnp.minimum(ts[i], jnp.maximum(ts[i - 1], d)))
                ts = tuple(new)
            return ts

        ts = lax.fori_loop(0, _M // 4, body, init)
        tot = ts[0]
        for i in range(1, _K):
            tot = tot + ts[i]
        acc_v[pl.ds(g * 16, 16)] = tot + jnp.float32(_K) * qn

    pltpu.sync_copy(acc_v, out_hbm.at[wid, 0])


def _sc_align(XhT, XT):
    xh3, x3 = lax.optimization_barrier(
        (XhT.reshape(_B * 3, 1, _N), XT.reshape(_B * 3, 1, _M)))
    mesh = plsc.VectorSubcoreMesh(core_axis_name="c", subcore_axis_name="s")
    run = pl.kernel(
        _sc_align_body,
        mesh=mesh,
        compiler_params=pltpu.CompilerParams(needs_layout_passes=False),
        out_type=jax.ShapeDtypeStruct((_NSUB, 1, _QPW), jnp.float32),
        scratch_types=[
            pltpu.VMEM((_QPW,), jnp.float32),
            pltpu.VMEM((_QPW,), jnp.float32),
            pltpu.VMEM((_QPW,), jnp.float32),
            pltpu.VMEM((_M,), jnp.float32),
            pltpu.VMEM((_M,), jnp.float32),
            pltpu.VMEM((_M,), jnp.float32),
            pltpu.VMEM((_M,), jnp.float32),
            pltpu.VMEM((_QPW,), jnp.float32),
            pltpu.SemaphoreType.DMA,
        ],
    )
    return run(xh3, x3)


def _trace3(A, B3):
    P = lax.dot_general(A, B3, (((1,), (0,)), ((), ())))
    eye = (lax.broadcasted_iota(jnp.int32, (3, 3), 0)
           == lax.broadcasted_iota(jnp.int32, (3, 3), 1))
    return jnp.sum(jnp.where(eye, P, 0.0))


def _tc_body(Yr_ref, Ya_ref, Rp_ref, tp_ref, Rg_ref, tg_ref,
             Xhr_ref, Xha_ref, dlr_ref, dla_ref, out_ref):
    t = pl.program_id(1)
    f32 = jnp.float32
    i32 = jnp.int32

    yrow = Yr_ref[0]
    yall = Ya_ref[0]
    Rp = Rp_ref[0]
    tp = tp_ref[0]
    Rg = Rg_ref[0]
    tg = tg_ref[0]
    xh_r = Xhr_ref[0]
    xh_a = Xha_ref[0]
    de_r = dlr_ref[0]
    de_a = dla_ref[0]

    mm = (((1,), (0,)), ((), ()))
    cT = (((0,), (0,)), ((), ()))

    yrig_r = lax.dot_general(Rp, yrow, mm) + tp
    yrig_a = lax.dot_general(Rp, yall, mm) + tp

    nr = jnp.sum(yrig_r * yrig_r, axis=0, keepdims=True)
    na = jnp.sum(yrig_a * yrig_a, axis=0, keepdims=True)
    ones_r = jnp.ones((1, _TILE), f32)
    ones_a = jnp.ones((1, _N), f32)
    U = jnp.concatenate([-2.0 * yrig_r, nr, ones_r], axis=0)
    V = jnp.concatenate([yrig_a, ones_a, na], axis=0)
    d = lax.dot_general(U, V, cT)

    row_id = t * _TILE + lax.broadcasted_iota(i32, (_TILE, _N), 0)
    col_id = lax.broadcasted_iota(i32, (_TILE, _N), 1)
    d = jnp.where(row_id == col_id, _SELF, d)

    m = jnp.min(d, axis=1, keepdims=True)
    for _ in range(_K - 1):
        m = jnp.min(jnp.where(d <= m, _INF, d), axis=1, keepdims=True)
    selmask = jnp.where(d <= m, 1.0, 0.0).astype(f32)

    D_a = xh_a - yrig_a
    D2_a = jnp.sum(D_a * D_a, axis=0, keepdims=True)
    F = jnp.concatenate([D_a, D2_a, de_a], axis=0)
    sel = lax.dot_general(selmask, F, (((1,), (1,)), ((), ())))
    S1 = sel[:, 0:3]
    S2 = sel[:, 3:4]
    Sd = sel[:, 4:7]

    D_r = xh_r - yrig_r
    deform_s = (jnp.sum(S2) - 2.0 * _trace3(D_r, S1)
                + f32(_K) * jnp.sum(D_r * D_r))
    disp_s = jnp.sum(de_r * de_r)
    lap_s = (disp_s - (2.0 / _K) * _trace3(de_r, Sd)
             + (1.0 / (_K * _K)) * jnp.sum(Sd * Sd))

    E = lax.dot_general(Rp - Rg, yrow, mm) + (tp - tg)
    rmse_s = jnp.sum(E * E)

    Rd = lax.dot_general(Rp, Rg, cT)
    eye = (lax.broadcasted_iota(i32, (3, 3), 0)
           == lax.broadcasted_iota(i32, (3, 3), 1))
    tr = jnp.sum(jnp.where(eye, Rd, 0.0))
    dtr = tp - tg
    trans_sq = jnp.sum(dtr * dtr)
    gate = jnp.where(t == 0, f32(1.0), f32(0.0))

    lane = lax.broadcasted_iota(i32, (1, 1, 128), 2)
    vals = (jnp.where(lane == 1, deform_s, 0.0)
            + jnp.where(lane == 2, lap_s, 0.0)
            + jnp.where(lane == 3, disp_s, 0.0)
            + jnp.where(lane == 4, rmse_s, 0.0)
            + jnp.where(lane == 5, gate * tr, 0.0)
            + jnp.where(lane == 6, gate * trans_sq, 0.0))

    @pl.when(t == 0)
    def _init():
        out_ref[...] = jnp.zeros_like(out_ref)

    out_ref[...] += vals


def kernel(Y, X, R_pred, t_pred, R_gt, t_gt, X_hat, delta):
    f32 = jnp.float32
    YT = jnp.swapaxes(Y, 1, 2)
    XT = jnp.swapaxes(X, 1, 2)
    XhT = jnp.swapaxes(X_hat, 1, 2)
    dlT = jnp.swapaxes(delta, 1, 2)
    tp3 = t_pred.reshape(_B, 3, 1).astype(f32)
    tg3 = t_gt.reshape(_B, 3, 1).astype(f32)

    align_parts = _sc_align(XhT, XT)

    rows = lambda b, t: (b, 0, t)
    full = lambda b, t: (b, 0, 0)

    tc_call = pl.pallas_call(
        _tc_body,
        grid=(_B, _T),
        in_specs=[
            pl.BlockSpec((1, 3, _TILE), rows),
            pl.BlockSpec((1, 3, _N), full),
            pl.BlockSpec((1, 3, 3), full),
            pl.BlockSpec((1, 3, 1), full),
            pl.BlockSpec((1, 3, 3), full),
            pl.BlockSpec((1, 3, 1), full),
            pl.BlockSpec((1, 3, _TILE), rows),
            pl.BlockSpec((1, 3, _N), full),
            pl.BlockSpec((1, 3, _TILE), rows),
            pl.BlockSpec((1, 3, _N), full),
        ],
        out_specs=pl.BlockSpec((1, 1, 128), full),
        out_shape=jax.ShapeDtypeStruct((_B, 1, 128), f32),
    )
    out = tc_call(YT, YT, R_pred, tp3, R_gt, tg3, XhT, XhT, dlT, dlT)

    o = out[:, 0, :]
    NK = f32(_N * _K)
    L_align_mean = jnp.sum(align_parts) / f32(_B * _N * _K)
    L_deform = o[:, 1] / NK
    L_lap = o[:, 2] / f32(_N)
    L_disp = o[:, 3] / f32(_N)
    L_rmse = jnp.sqrt(o[:, 4] / f32(_N))
    tr = o[:, 5]
    trans_sq = o[:, 6]
    c = jnp.clip((tr - 1.0) / 2.0, -1.0 + 1e-07, 1.0 - 1e-07)
    L_rot = jnp.arccos(c)
    L_trans = jnp.sqrt(trans_sq)
    total = (L_rot + L_trans + L_rmse
             + 0.01 * L_disp + 0.1 * L_deform + 0.1 * L_lap)
    return total.mean() + L_align_mean


# speedup vs baseline: 1.0131x; 1.0131x over previous
import functools
import jax
import jax.numpy as jnp
from jax import lax
from jax.experimental import pallas as pl
from jax.experimental.pallas import tpu as pltpu
from jax.experimental.pallas import tpu_sc as plsc

_B, _N, _M, _K = 2, 2048, 1024, 5
_TILE = 1024
_T = _N // _TILE
_INF = 3.0e38
_SELF = 1.0e38

_NSUB = 32
_QPW = (_B * _N) // _NSUB
_GRP = _QPW // 16


def _sc_align_body(xh_hbm, x_hbm, out_hbm, qx_v, qy_v, qz_v,
                   cx_v, cy_v, cz_v, cn_v, acc_v, sem):
    c = lax.axis_index("c")
    s = lax.axis_index("s")
    wid = s * 2 + c
    b = wid // (_NSUB // _B)
    chunk = wid % (_NSUB // _B)
    q0 = pl.multiple_of(chunk * _QPW, 8)

    cps = [
        pltpu.async_copy(xh_hbm.at[b * 3 + 0, 0, pl.ds(q0, _QPW)], qx_v, sem),
        pltpu.async_copy(xh_hbm.at[b * 3 + 1, 0, pl.ds(q0, _QPW)], qy_v, sem),
        pltpu.async_copy(xh_hbm.at[b * 3 + 2, 0, pl.ds(q0, _QPW)], qz_v, sem),
        pltpu.async_copy(x_hbm.at[b * 3 + 0, 0], cx_v, sem),
        pltpu.async_copy(x_hbm.at[b * 3 + 1, 0], cy_v, sem),
        pltpu.async_copy(x_hbm.at[b * 3 + 2, 0], cz_v, sem),
    ]
    for cp in cps:
        cp.wait()

    def prep(k, carry):
        sl = pl.ds(k * 16, 16)
        cx = cx_v[sl]
        cy = cy_v[sl]
        cz = cz_v[sl]
        cn_v[sl] = (cx * cx + cy * cy) + cz * cz
        cx_v[sl] = -2.0 * cx
        cy_v[sl] = -2.0 * cy
        cz_v[sl] = -2.0 * cz
        return carry

    lax.fori_loop(0, _M // 16, prep, 0)

    for g in range(_GRP):
        qx = qx_v[pl.ds(g * 16, 16)]
        qy = qy_v[pl.ds(g * 16, 16)]
        qz = qz_v[pl.ds(g * 16, 16)]
        qn = (qx * qx + qy * qy) + qz * qz

        init = tuple(jnp.full((16,), _INF, jnp.float32) for _ in range(_K))

        def body(jj, ts, qx=qx, qy=qy, qz=qz):
            for u in range(2):
                j = jj * 2 + u
                idx = jnp.full((16,), j, jnp.int32)
                gx = plsc.load_gather(cx_v, [idx])
                gy = plsc.load_gather(cy_v, [idx])
                gz = plsc.load_gather(cz_v, [idx])
                gn = plsc.load_gather(cn_v, [idx])
                d = (gn + qx * gx) + (qy * gy + qz * gz)
                new = [jnp.minimum(ts[0], d)]
                for i in range(1, _K):
                    new.append(jnp.minimum(ts[i], jnp.maximum(ts[i - 1], d)))
                ts = tuple(new)
            return ts

        ts = lax.fori_loop(0, _M // 2, body, init)
        tot = ts[0]
        for i in range(1, _K):
            tot = tot + ts[i]
        acc_v[pl.ds(g * 16, 16)] = tot + jnp.float32(_K) * qn

    pltpu.sync_copy(acc_v, out_hbm.at[wid, 0])


def _sc_align(XhT, XT):
    xh3 = XhT.reshape(_B * 3, 1, _N)
    x3 = XT.reshape(_B * 3, 1, _M)
    mesh = plsc.VectorSubcoreMesh(core_axis_name="c", subcore_axis_name="s")
    run = pl.kernel(
        _sc_align_body,
        mesh=mesh,
        compiler_params=pltpu.CompilerParams(needs_layout_passes=False),
        out_type=jax.ShapeDtypeStruct((_NSUB, 1, _QPW), jnp.float32),
        scratch_types=[
            pltpu.VMEM((_QPW,), jnp.float32),
            pltpu.VMEM((_QPW,), jnp.float32),
            pltpu.VMEM((_QPW,), jnp.float32),
            pltpu.VMEM((_M,), jnp.float32),
            pltpu.VMEM((_M,), jnp.float32),
            pltpu.VMEM((_M,), jnp.float32),
            pltpu.VMEM((_M,), jnp.float32),
            pltpu.VMEM((_QPW,), jnp.float32),
            pltpu.SemaphoreType.DMA,
        ],
    )
    return run(xh3, x3)


def _trace3(A, B3):
    P = lax.dot_general(A, B3, (((1,), (0,)), ((), ())))
    eye = (lax.broadcasted_iota(jnp.int32, (3, 3), 0)
           == lax.broadcasted_iota(jnp.int32, (3, 3), 1))
    return jnp.sum(jnp.where(eye, P, 0.0))


def _tc_body(Yr_ref, Ya_ref, Rp_ref, tp_ref, Rg_ref, tg_ref,
             Xhr_ref, Xha_ref, dlr_ref, dla_ref, out_ref):
    t = pl.program_id(1)
    f32 = jnp.float32
    i32 = jnp.int32

    yrow = Yr_ref[0]
    yall = Ya_ref[0]
    Rp = Rp_ref[0]
    tp = tp_ref[0]
    Rg = Rg_ref[0]
    tg = tg_ref[0]
    xh_r = Xhr_ref[0]
    xh_a = Xha_ref[0]
    de_r = dlr_ref[0]
    de_a = dla_ref[0]

    mm = (((1,), (0,)), ((), ()))
    cT = (((0,), (0,)), ((), ()))

    yrig_r = lax.dot_general(Rp, yrow, mm) + tp
    yrig_a = lax.dot_general(Rp, yall, mm) + tp

    nr = jnp.sum(yrig_r * yrig_r, axis=0, keepdims=True)
    na = jnp.sum(yrig_a * yrig_a, axis=0, keepdims=True)
    ones_r = jnp.ones((1, _TILE), f32)
    ones_a = jnp.ones((1, _N), f32)
    U = jnp.concatenate([-2.0 * yrig_r, nr, ones_r], axis=0)
    V = jnp.concatenate([yrig_a, ones_a, na], axis=0)
    d = lax.dot_general(U, V, cT)

    row_id = t * _TILE + lax.broadcasted_iota(i32, (_TILE, _N), 0)
    col_id = lax.broadcasted_iota(i32, (_TILE, _N), 1)
    d = jnp.where(row_id == col_id, _SELF, d)

    m = jnp.min(d, axis=1, keepdims=True)
    for _ in range(_K - 1):
        m = jnp.min(jnp.where(d <= m, _INF, d), axis=1, keepdims=True)
    selmask = jnp.where(d <= m, 1.0, 0.0).astype(f32)

    D_a = xh_a - yrig_a
    D2_a = jnp.sum(D_a * D_a, axis=0, keepdims=True)
    F = jnp.concatenate([D_a, D2_a, de_a], axis=0)
    sel = lax.dot_general(selmask, F, (((1,), (1,)), ((), ())))
    S1 = sel[:, 0:3]
    S2 = sel[:, 3:4]
    Sd = sel[:, 4:7]

    D_r = xh_r - yrig_r
    deform_s = (jnp.sum(S2) - 2.0 * _trace3(D_r, S1)
                + f32(_K) * jnp.sum(D_r * D_r))
    disp_s = jnp.sum(de_r * de_r)
    lap_s = (disp_s - (2.0 / _K) * _trace3(de_r, Sd)
             + (1.0 / (_K * _K)) * jnp.sum(Sd * Sd))

    E = lax.dot_general(Rp - Rg, yrow, mm) + (tp - tg)
    rmse_s = jnp.sum(E * E)

    Rd = lax.dot_general(Rp, Rg, cT)
    eye = (lax.broadcasted_iota(i32, (3, 3), 0)
           == lax.broadcasted_iota(i32, (3, 3), 1))
    tr = jnp.sum(jnp.where(eye, Rd, 0.0))
    dtr = tp - tg
    trans_sq = jnp.sum(dtr * dtr)
    gate = jnp.where(t == 0, f32(1.0), f32(0.0))

    lane = lax.broadcasted_iota(i32, (1, 1, 128), 2)
    vals = (jnp.where(lane == 1, deform_s, 0.0)
            + jnp.where(lane == 2, lap_s, 0.0)
            + jnp.where(lane == 3, disp_s, 0.0)
            + jnp.where(lane == 4, rmse_s, 0.0)
            + jnp.where(lane == 5, gate * tr, 0.0)
            + jnp.where(lane == 6, gate * trans_sq, 0.0))

    @pl.when(t == 0)
    def _init():
        out_ref[...] = jnp.zeros_like(out_ref)

    out_ref[...] += vals


def kernel(Y, X, R_pred, t_pred, R_gt, t_gt, X_hat, delta):
    f32 = jnp.float32
    YT = jnp.swapaxes(Y, 1, 2)
    XT = jnp.swapaxes(X, 1, 2)
    XhT = jnp.swapaxes(X_hat, 1, 2)
    dlT = jnp.swapaxes(delta, 1, 2)
    tp3 = t_pred.reshape(_B, 3, 1).astype(f32)
    tg3 = t_gt.reshape(_B, 3, 1).astype(f32)

    align_parts = _sc_align(XhT, XT)

    rows = lambda b, t: (b, 0, t)
    full = lambda b, t: (b, 0, 0)

    tc_call = pl.pallas_call(
        _tc_body,
        grid=(_B, _T),
        in_specs=[
            pl.BlockSpec((1, 3, _TILE), rows),
            pl.BlockSpec((1, 3, _N), full),
            pl.BlockSpec((1, 3, 3), full),
            pl.BlockSpec((1, 3, 1), full),
            pl.BlockSpec((1, 3, 3), full),
            pl.BlockSpec((1, 3, 1), full),
            pl.BlockSpec((1, 3, _TILE), rows),
            pl.BlockSpec((1, 3, _N), full),
            pl.BlockSpec((1, 3, _TILE), rows),
            pl.BlockSpec((1, 3, _N), full),
        ],
        out_specs=pl.BlockSpec((1, 1, 128), full),
        out_shape=jax.ShapeDtypeStruct((_B, 1, 128), f32),
    )
    out = tc_call(YT, YT, R_pred, tp3, R_gt, tg3, XhT, XhT, dlT, dlT)

    o = out[:, 0, :]
    NK = f32(_N * _K)
    L_align_mean = jnp.sum(align_parts) / f32(_B * _N * _K)
    L_deform = o[:, 1] / NK
    L_lap = o[:, 2] / f32(_N)
    L_disp = o[:, 3] / f32(_N)
    L_rmse = jnp.sqrt(o[:, 4] / f32(_N))
    tr = o[:, 5]
    trans_sq = o[:, 6]
    c = jnp.clip((tr - 1.0) / 2.0, -1.0 + 1e-07, 1.0 - 1e-07)
    L_rot = jnp.arccos(c)
    L_trans = jnp.sqrt(trans_sq)
    total = (L_rot + L_trans + L_rmse
             + 0.01 * L_disp + 0.1 * L_deform + 0.1 * L_lap)
    return total.mean() + L_align_mean


# speedup vs baseline: 1.0875x; 1.0734x over previous
import functools
import jax
import jax.numpy as jnp
from jax import lax
from jax.experimental import pallas as pl
from jax.experimental.pallas import tpu as pltpu
from jax.experimental.pallas import tpu_sc as plsc

_B, _N, _M, _K = 2, 2048, 1024, 5
_TILE = 2048
_T = _N // _TILE
_INF = 3.0e38
_SELF = 1.0e38

_NSUB = 32
_QPW = (_B * _N) // _NSUB
_GRP = _QPW // 16


def _sc_align_body(xh_hbm, x_hbm, out_hbm, qx_v, qy_v, qz_v,
                   cx_v, cy_v, cz_v, cn_v, acc_v, sem):
    c = lax.axis_index("c")
    s = lax.axis_index("s")
    wid = s * 2 + c
    b = wid // (_NSUB // _B)
    chunk = wid % (_NSUB // _B)
    q0 = pl.multiple_of(chunk * _QPW, 8)

    cps = [
        pltpu.async_copy(xh_hbm.at[b * 3 + 0, 0, pl.ds(q0, _QPW)], qx_v, sem),
        pltpu.async_copy(xh_hbm.at[b * 3 + 1, 0, pl.ds(q0, _QPW)], qy_v, sem),
        pltpu.async_copy(xh_hbm.at[b * 3 + 2, 0, pl.ds(q0, _QPW)], qz_v, sem),
        pltpu.async_copy(x_hbm.at[b * 3 + 0, 0], cx_v, sem),
        pltpu.async_copy(x_hbm.at[b * 3 + 1, 0], cy_v, sem),
        pltpu.async_copy(x_hbm.at[b * 3 + 2, 0], cz_v, sem),
    ]
    for cp in cps:
        cp.wait()

    def prep(k, carry):
        sl = pl.ds(k * 16, 16)
        cx = cx_v[sl]
        cy = cy_v[sl]
        cz = cz_v[sl]
        cn_v[sl] = (cx * cx + cy * cy) + cz * cz
        cx_v[sl] = -2.0 * cx
        cy_v[sl] = -2.0 * cy
        cz_v[sl] = -2.0 * cz
        return carry

    lax.fori_loop(0, _M // 16, prep, 0)

    for g in range(_GRP):
        qx = qx_v[pl.ds(g * 16, 16)]
        qy = qy_v[pl.ds(g * 16, 16)]
        qz = qz_v[pl.ds(g * 16, 16)]
        qn = (qx * qx + qy * qy) + qz * qz

        init = tuple(jnp.full((16,), _INF, jnp.float32) for _ in range(_K))

        def body(jj, ts, qx=qx, qy=qy, qz=qz):
            for u in range(2):
                j = jj * 2 + u
                idx = jnp.full((16,), j, jnp.int32)
                gx = plsc.load_gather(cx_v, [idx])
                gy = plsc.load_gather(cy_v, [idx])
                gz = plsc.load_gather(cz_v, [idx])
                gn = plsc.load_gather(cn_v, [idx])
                d = (gn + qx * gx) + (qy * gy + qz * gz)
                new = [jnp.minimum(ts[0], d)]
                for i in range(1, _K):
                    new.append(jnp.minimum(ts[i], jnp.maximum(ts[i - 1], d)))
                ts = tuple(new)
            return ts

        ts = lax.fori_loop(0, _M // 2, body, init)
        tot = ts[0]
        for i in range(1, _K):
            tot = tot + ts[i]
        acc_v[pl.ds(g * 16, 16)] = tot + jnp.float32(_K) * qn

    pltpu.sync_copy(acc_v, out_hbm.at[wid, 0])


def _sc_align(XhT, XT):
    xh3 = XhT.reshape(_B * 3, 1, _N)
    x3 = XT.reshape(_B * 3, 1, _M)
    mesh = plsc.VectorSubcoreMesh(core_axis_name="c", subcore_axis_name="s")
    run = pl.kernel(
        _sc_align_body,
        mesh=mesh,
        compiler_params=pltpu.CompilerParams(needs_layout_passes=False),
        out_type=jax.ShapeDtypeStruct((_NSUB, 1, _QPW), jnp.float32),
        scratch_types=[
            pltpu.VMEM((_QPW,), jnp.float32),
            pltpu.VMEM((_QPW,), jnp.float32),
            pltpu.VMEM((_QPW,), jnp.float32),
            pltpu.VMEM((_M,), jnp.float32),
            pltpu.VMEM((_M,), jnp.float32),
            pltpu.VMEM((_M,), jnp.float32),
            pltpu.VMEM((_M,), jnp.float32),
            pltpu.VMEM((_QPW,), jnp.float32),
            pltpu.SemaphoreType.DMA,
        ],
    )
    return run(xh3, x3)


def _trace3(A, B3):
    P = lax.dot_general(A, B3, (((1,), (0,)), ((), ())))
    eye = (lax.broadcasted_iota(jnp.int32, (3, 3), 0)
           == lax.broadcasted_iota(jnp.int32, (3, 3), 1))
    return jnp.sum(jnp.where(eye, P, 0.0))


def _tc_body(Yr_ref, Ya_ref, Rp_ref, tp_ref, Rg_ref, tg_ref,
             Xhr_ref, Xha_ref, dlr_ref, dla_ref, out_ref):
    t = pl.program_id(1)
    f32 = jnp.float32
    i32 = jnp.int32

    yrow = Yr_ref[0]
    yall = Ya_ref[0]
    Rp = Rp_ref[0]
    tp = tp_ref[0]
    Rg = Rg_ref[0]
    tg = tg_ref[0]
    xh_r = Xhr_ref[0]
    xh_a = Xha_ref[0]
    de_r = dlr_ref[0]
    de_a = dla_ref[0]

    mm = (((1,), (0,)), ((), ()))
    cT = (((0,), (0,)), ((), ()))

    yrig_r = lax.dot_general(Rp, yrow, mm) + tp
    yrig_a = lax.dot_general(Rp, yall, mm) + tp

    nr = jnp.sum(yrig_r * yrig_r, axis=0, keepdims=True)
    na = jnp.sum(yrig_a * yrig_a, axis=0, keepdims=True)
    ones_r = jnp.ones((1, _TILE), f32)
    ones_a = jnp.ones((1, _N), f32)
    U = jnp.concatenate([-2.0 * yrig_r, nr, ones_r], axis=0)
    V = jnp.concatenate([yrig_a, ones_a, na], axis=0)
    d = lax.dot_general(U, V, cT)

    row_id = t * _TILE + lax.broadcasted_iota(i32, (_TILE, _N), 0)
    col_id = lax.broadcasted_iota(i32, (_TILE, _N), 1)
    d = jnp.where(row_id == col_id, _SELF, d)

    m = jnp.min(d, axis=1, keepdims=True)
    for _ in range(_K - 1):
        m = jnp.min(jnp.where(d <= m, _INF, d), axis=1, keepdims=True)
    selmask = jnp.where(d <= m, 1.0, 0.0).astype(f32)

    D_a = xh_a - yrig_a
    D2_a = jnp.sum(D_a * D_a, axis=0, keepdims=True)
    F = jnp.concatenate([D_a, D2_a, de_a], axis=0)
    sel = lax.dot_general(selmask, F, (((1,), (1,)), ((), ())))
    S1 = sel[:, 0:3]
    S2 = sel[:, 3:4]
    Sd = sel[:, 4:7]

    D_r = xh_r - yrig_r
    deform_s = (jnp.sum(S2) - 2.0 * _trace3(D_r, S1)
                + f32(_K) * jnp.sum(D_r * D_r))
    disp_s = jnp.sum(de_r * de_r)
    lap_s = (disp_s - (2.0 / _K) * _trace3(de_r, Sd)
             + (1.0 / (_K * _K)) * jnp.sum(Sd * Sd))

    E = lax.dot_general(Rp - Rg, yrow, mm) + (tp - tg)
    rmse_s = jnp.sum(E * E)

    Rd = lax.dot_general(Rp, Rg, cT)
    eye = (lax.broadcasted_iota(i32, (3, 3), 0)
           == lax.broadcasted_iota(i32, (3, 3), 1))
    tr = jnp.sum(jnp.where(eye, Rd, 0.0))
    dtr = tp - tg
    trans_sq = jnp.sum(dtr * dtr)
    gate = jnp.where(t == 0, f32(1.0), f32(0.0))

    lane = lax.broadcasted_iota(i32, (1, 1, 128), 2)
    vals = (jnp.where(lane == 1, deform_s, 0.0)
            + jnp.where(lane == 2, lap_s, 0.0)
            + jnp.where(lane == 3, disp_s, 0.0)
            + jnp.where(lane == 4, rmse_s, 0.0)
            + jnp.where(lane == 5, gate * tr, 0.0)
            + jnp.where(lane == 6, gate * trans_sq, 0.0))

    @pl.when(t == 0)
    def _init():
        out_ref[...] = jnp.zeros_like(out_ref)

    out_ref[...] += vals


def kernel(Y, X, R_pred, t_pred, R_gt, t_gt, X_hat, delta):
    f32 = jnp.float32
    YT = jnp.swapaxes(Y, 1, 2)
    XT = jnp.swapaxes(X, 1, 2)
    XhT = jnp.swapaxes(X_hat, 1, 2)
    dlT = jnp.swapaxes(delta, 1, 2)
    tp3 = t_pred.reshape(_B, 3, 1).astype(f32)
    tg3 = t_gt.reshape(_B, 3, 1).astype(f32)

    align_parts = _sc_align(XhT, XT)

    rows = lambda b, t: (b, 0, t)
    full = lambda b, t: (b, 0, 0)

    tc_call = pl.pallas_call(
        _tc_body,
        grid=(_B, _T),
        in_specs=[
            pl.BlockSpec((1, 3, _TILE), rows),
            pl.BlockSpec((1, 3, _N), full),
            pl.BlockSpec((1, 3, 3), full),
            pl.BlockSpec((1, 3, 1), full),
            pl.BlockSpec((1, 3, 3), full),
            pl.BlockSpec((1, 3, 1), full),
            pl.BlockSpec((1, 3, _TILE), rows),
            pl.BlockSpec((1, 3, _N), full),
            pl.BlockSpec((1, 3, _TILE), rows),
            pl.BlockSpec((1, 3, _N), full),
        ],
        out_specs=pl.BlockSpec((1, 1, 128), full),
        out_shape=jax.ShapeDtypeStruct((_B, 1, 128), f32),
    )
    out = tc_call(YT, YT, R_pred, tp3, R_gt, tg3, XhT, XhT, dlT, dlT)

    o = out[:, 0, :]
    NK = f32(_N * _K)
    L_align_mean = jnp.sum(align_parts) / f32(_B * _N * _K)
    L_deform = o[:, 1] / NK
    L_lap = o[:, 2] / f32(_N)
    L_disp = o[:, 3] / f32(_N)
    L_rmse = jnp.sqrt(o[:, 4] / f32(_N))
    tr = o[:, 5]
    trans_sq = o[:, 6]
    c = jnp.clip((tr - 1.0) / 2.0, -1.0 + 1e-07, 1.0 - 1e-07)
    L_rot = jnp.arccos(c)
    L_trans = jnp.sqrt(trans_sq)
    total = (L_rot + L_trans + L_rmse
             + 0.01 * L_disp + 0.1 * L_deform + 0.1 * L_lap)
    return total.mean() + L_align_mean
